# baseline (speedup 1.0000x reference)
import functools

import jax
import jax.numpy as jnp
from jax import lax
from jax.experimental import pallas as pl
from jax.experimental.pallas import tpu as pltpu
from jax.experimental.pallas import tpu_sc as plsc

L = 16
NC = 2
NS = 16
NW = NC * NS
N = 2 * 4096 * 4096
PER_W = N // NW
QMAX = 127.0
MAGIC = 1.5 * 2 ** 23

CHUNK1 = 32768
NCHUNK1 = PER_W // CHUNK1
CHUNK2 = 16384
NCHUNK2 = PER_W // CHUNK2
U = 8

_mesh = plsc.VectorSubcoreMesh(core_axis_name="c", subcore_axis_name="s")


@functools.partial(
    pl.kernel,
    mesh=_mesh,
    out_type=jax.ShapeDtypeStruct((NW, L), jnp.float32),
    scratch_types=[
        pltpu.VMEM((CHUNK1,), jnp.float32),
        pltpu.VMEM((CHUNK1,), jnp.float32),
        pltpu.VMEM((L,), jnp.float32),
        pltpu.SemaphoreType.DMA,
        pltpu.SemaphoreType.DMA,
    ],
)
def _absmax_kernel(x_hbm, out_hbm, buf0, buf1, accb, sem0, sem1):
    wid = lax.axis_index("s") * NC + lax.axis_index("c")
    base = wid * PER_W
    bufs = (buf0, buf1)
    sems = (sem0, sem1)
    copies = [
        pltpu.async_copy(x_hbm.at[pl.ds(base + i * CHUNK1, CHUNK1)], bufs[i], sems[i])
        for i in range(2)
    ]
    acc = jnp.zeros((L,), jnp.float32)
    for i in range(NCHUNK1):
        b = i % 2
        copies[b].wait()
        buf = bufs[b]

        def body(j, a, buf=buf):
            for u in range(U):
                a = jnp.maximum(a, jnp.abs(buf[pl.ds((j * U + u) * L, L)]))
            return a

        acc = lax.fori_loop(0, CHUNK1 // (L * U), body, acc)
        nxt = i + 2
        if nxt < NCHUNK1:
            copies[b] = pltpu.async_copy(
                x_hbm.at[pl.ds(base + nxt * CHUNK1, CHUNK1)], bufs[b], sems[b]
            )
    accb[...] = acc
    pltpu.sync_copy(accb, out_hbm.at[wid])


@functools.partial(
    pl.kernel,
    mesh=_mesh,
    out_type=jax.ShapeDtypeStruct((N,), jnp.float32),
    scratch_types=[
        pltpu.VMEM((CHUNK2,), jnp.float32),
        pltpu.VMEM((CHUNK2,), jnp.float32),
        pltpu.VMEM((CHUNK2,), jnp.float32),
        pltpu.VMEM((CHUNK2,), jnp.float32),
        pltpu.VMEM((NW, L), jnp.float32),
        pltpu.SemaphoreType.DMA,
        pltpu.SemaphoreType.DMA,
        pltpu.SemaphoreType.DMA,
        pltpu.SemaphoreType.DMA,
    ],
)
def _quant_kernel(x_hbm, pmax_hbm, out_hbm, in0, in1, ob0, ob1, pbuf,
                  isem0, isem1, osem0, osem1):
    wid = lax.axis_index("s") * NC + lax.axis_index("c")
    base = wid * PER_W
    ibufs = (in0, in1)
    isems = (isem0, isem1)
    obufs = (ob0, ob1)
    osems = (osem0, osem1)

    pltpu.sync_copy(pmax_hbm, pbuf)
    v = pbuf[0]
    for i in range(1, NW):
        v = jnp.maximum(v, pbuf[i])
    absmax = v[0]
    for i in range(1, L):
        absmax = jnp.maximum(absmax, v[i])
    svec = jnp.full((L,), absmax, jnp.float32) * (1.0 / QMAX)
    yi = 0x7EB53567 - lax.bitcast_convert_type(svec, jnp.int32)
    y = lax.bitcast_convert_type(yi, jnp.float32)
    for _ in range(4):
        y = y * (2.0 - svec * y)
    scale = svec
    inv = y

    copies = [
        pltpu.async_copy(x_hbm.at[pl.ds(base + i * CHUNK2, CHUNK2)], ibufs[i], isems[i])
        for i in range(2)
    ]
    ocopies = [None, None]
    for i in range(NCHUNK2):
        b = i % 2
        copies[b].wait()
        ibuf, obuf = ibufs[b], obufs[b]
        if ocopies[b] is not None:
            ocopies[b].wait()

        def body(j, carry, ibuf=ibuf, obuf=obuf):
            for u in range(U):
                off = (j * U + u) * L
                t = ibuf[pl.ds(off, L)] * inv
                q = (t + MAGIC) - MAGIC
                obuf[pl.ds(off, L)] = q * scale
            return carry

        lax.fori_loop(0, CHUNK2 // (L * U), body, 0)
        ocopies[b] = pltpu.async_copy(
            obuf, out_hbm.at[pl.ds(base + i * CHUNK2, CHUNK2)], osems[b]
        )
        nxt = i + 2
        if nxt < NCHUNK2:
            copies[b] = pltpu.async_copy(
                x_hbm.at[pl.ds(base + nxt * CHUNK2, CHUNK2)], ibufs[b], isems[b]
            )
    for b in range(2):
        if ocopies[b] is not None:
            ocopies[b].wait()


def kernel(tensor):
    x = tensor.reshape(-1)
    pmax = _absmax_kernel(x)
    dq = _quant_kernel(x, pmax)
    return dq.reshape(tensor.shape)
